# Initial kernel scaffold; baseline (speedup 1.0000x reference)
import jax
import jax.numpy as jnp
from jax.experimental import pallas as pl


def kernel(x, edge_index, edge_attr, batch, Wu1, bu1, Wu2, bu2, We1, be1, Wm1, bm1, Wn1, bn1, g1, b1, We2, be2, Wm2, bm2, Wn2, bn2, g2, b2):
    raise NotImplementedError("write your pallas kernel here")


# speedup vs baseline: 1.7556x; 1.7556x over previous
import functools

import jax
import jax.numpy as jnp
from jax import lax
from jax.experimental import pallas as pl
from jax.experimental.pallas import tpu as pltpu
from jax.experimental.pallas import tpu_sc as plsc

_NC = 2
_NS = 16
_L = 16

_T_STEPS = 3
_NUM_GRAPHS = 64


def _sc_gather(table, idx2d):
    R, D = table.shape
    nrow = idx2d.shape[0]
    E = nrow * 128
    rows_pt = nrow // (_NC * _NS)
    ept = rows_pt * 128
    mesh = plsc.VectorSubcoreMesh(core_axis_name="c", subcore_axis_name="s")

    @functools.partial(
        pl.kernel, mesh=mesh,
        out_type=jax.ShapeDtypeStruct((E, D), jnp.float32),
        scratch_types=[
            pltpu.VMEM((ept, D), jnp.float32),
            pltpu.VMEM((rows_pt, 128), jnp.int32),
            pltpu.SemaphoreType.DMA,
        ],
    )
    def k(table_hbm, idx_hbm, out_hbm, buf, idxv, sem):
        c = lax.axis_index("c")
        s = lax.axis_index("s")
        w = c * _NS + s
        rb = w * rows_pt
        pltpu.sync_copy(idx_hbm.at[pl.ds(rb, rows_pt)], idxv)
        for j in range(rows_pt):
            pltpu.async_copy(table_hbm.at[idxv.at[j]],
                             buf.at[pl.ds(j * 128, 128)], sem).wait()
        pltpu.sync_copy(buf, out_hbm.at[pl.ds(w * ept, ept)])

    return k(table, idx2d)


def _sc_msg(h, tpos2d, tneg2d, gsrc2d, grev2d, H):
    E, D = h.shape
    ept_s = E // _NS
    srows = ept_s // 128
    ept_g = E // (_NC * _NS)
    grows = ept_g // 128
    BUF = ept_g
    brows = BUF // 128
    zpt = H // _NS
    mesh = plsc.VectorSubcoreMesh(core_axis_name="c", subcore_axis_name="s")

    @functools.partial(
        pl.kernel, mesh=mesh,
        out_type=[jax.ShapeDtypeStruct((E, D), jnp.float32),
                  jax.ShapeDtypeStruct((E, D), jnp.float32)],
        scratch_types=[
            pltpu.VMEM_SHARED((H, D), jnp.float32),
            pltpu.VMEM((BUF, D), jnp.float32),
            pltpu.VMEM((_L, D), jnp.float32),
            pltpu.VMEM((srows, 128), jnp.int32),
            pltpu.VMEM((srows, 128), jnp.int32),
            pltpu.VMEM((grows, 128), jnp.int32),
            pltpu.VMEM((grows, 128), jnp.int32),
            pltpu.SemaphoreType.DMA,
        ],
    )
    def k(h_hbm, tpos_hbm, tneg_hbm, gsrc_hbm, grev_hbm, mpos_hbm, mneg_hbm,
          table, buf, zbuf, tposv, tnegv, gsrcv, grevv, sem):
        c = lax.axis_index("c")
        s = lax.axis_index("s")
        for i in range(_L):
            for q in range(D // _L):
                zbuf[i, pl.ds(q * _L, _L)] = jnp.zeros((_L,), jnp.float32)
        zcopies = [
            pltpu.async_copy(zbuf, table.at[pl.ds(s * zpt + i * _L, _L)], sem)
            for i in range(zpt // _L)
        ]
        for zc in zcopies:
            zc.wait()
        plsc.subcore_barrier()
        pltpu.sync_copy(tpos_hbm.at[pl.ds(s * srows, srows)], tposv)
        pltpu.sync_copy(tneg_hbm.at[pl.ds(s * srows, srows)], tnegv)
        for ch in range(ept_s // BUF):
            eb = s * ept_s + ch * BUF
            pltpu.sync_copy(h_hbm.at[pl.ds(eb, BUF)], buf)
            for j in range(brows):
                pltpu.sync_copy(buf.at[pl.ds(j * 128, 128)],
                                table.at[tposv.at[ch * brows + j]], add=True)
            for j in range(brows):
                pltpu.sync_copy(buf.at[pl.ds(j * 128, 128)],
                                table.at[tnegv.at[ch * brows + j]], add=True)
        plsc.subcore_barrier()
        w = c * _NS + s
        gb = w * ept_g
        pltpu.sync_copy(gsrc_hbm.at[pl.ds(w * grows, grows)], gsrcv)
        pltpu.sync_copy(grev_hbm.at[pl.ds(w * grows, grows)], grevv)
        for j in range(grows):
            pltpu.async_copy(table.at[gsrcv.at[j]],
                             buf.at[pl.ds(j * 128, 128)], sem).wait()
        pltpu.sync_copy(buf, mpos_hbm.at[pl.ds(gb, ept_g)])
        for j in range(grows):
            pltpu.async_copy(table.at[grevv.at[j]],
                             buf.at[pl.ds(j * 128, 128)], sem).wait()
        pltpu.sync_copy(buf, mneg_hbm.at[pl.ds(gb, ept_g)])

    return k(h, tpos2d, tneg2d, gsrc2d, grev2d)


def _sc_nodemsg(h, tpos2d, N):
    E, D = h.shape
    ept_s = E // _NS
    srows = ept_s // 128
    zpt = N // _NS
    npc = N // _NC
    nps = npc // _NS
    mesh = plsc.VectorSubcoreMesh(core_axis_name="c", subcore_axis_name="s")

    @functools.partial(
        pl.kernel, mesh=mesh,
        out_type=jax.ShapeDtypeStruct((N, D), jnp.float32),
        scratch_types=[
            pltpu.VMEM_SHARED((N, D), jnp.float32),
            pltpu.VMEM((ept_s, D), jnp.float32),
            pltpu.VMEM((_L, D), jnp.float32),
            pltpu.VMEM((srows, 128), jnp.int32),
        ],
    )
    def k(h_hbm, tpos_hbm, out_hbm, table, buf, zbuf, tposv):
        c = lax.axis_index("c")
        s = lax.axis_index("s")
        for i in range(_L):
            for q in range(D // _L):
                zbuf[i, pl.ds(q * _L, _L)] = jnp.zeros((_L,), jnp.float32)
        for i in range(zpt // _L):
            pltpu.sync_copy(zbuf, table.at[pl.ds(s * zpt + i * _L, _L)])
        plsc.subcore_barrier()
        eb = s * ept_s
        pltpu.sync_copy(h_hbm.at[pl.ds(eb, ept_s)], buf)
        pltpu.sync_copy(tpos_hbm.at[pl.ds(s * srows, srows)], tposv)
        for j in range(srows):
            pltpu.sync_copy(buf.at[pl.ds(j * 128, 128)],
                            table.at[tposv.at[j]], add=True)
        plsc.subcore_barrier()
        rb = c * npc + s * nps
        pltpu.sync_copy(table.at[pl.ds(rb, nps)], buf.at[pl.ds(0, nps)])
        pltpu.sync_copy(buf.at[pl.ds(0, nps)], out_hbm.at[pl.ds(rb, nps)])

    return k(h, tpos2d)


def _relu(v):
    return jnp.maximum(v, 0.0)


def _elu(v):
    return jnp.where(v > 0, v, jnp.exp(jnp.minimum(v, 0.0)) - 1.0)


def _dot(a, b):
    return jax.lax.dot_general(a, b, (((1,), (0,)), ((), ())),
                               precision=jax.lax.Precision.HIGHEST,
                               preferred_element_type=jnp.float32)


def _tc_prep1(xs, ea, Wu1T, bu1, Wu2T, bu2, We1aT, We1bT, be1):
    E, D = xs.shape
    De = ea.shape[1]
    BE = 2048

    def body(xs_r, ea_r, wu1_r, bu1_r, wu2_r, bu2_r, wa_r, wb_r, be_r,
             h0_r, e2_r):
        e1 = _elu(_dot(ea_r[...], wu1_r[...]) + bu1_r[...])
        e2_r[...] = _elu(_dot(e1, wu2_r[...]) + bu2_r[...])
        h0_r[...] = _relu(_dot(xs_r[...], wa_r[...]) + _dot(e1, wb_r[...])
                          + be_r[...])

    full = lambda shape: pl.BlockSpec(shape, lambda i: (0, 0))
    return pl.pallas_call(
        body,
        grid=(E // BE,),
        in_specs=[
            pl.BlockSpec((BE, D), lambda i: (i, 0)),
            pl.BlockSpec((BE, De), lambda i: (i, 0)),
            full((De, De)), full((1, De)), full((De, De)), full((1, De)),
            full((D, D)), full((De, D)), full((1, D)),
        ],
        out_specs=[
            pl.BlockSpec((BE, D), lambda i: (i, 0)),
            pl.BlockSpec((BE, De), lambda i: (i, 0)),
        ],
        out_shape=[
            jax.ShapeDtypeStruct((E, D), jnp.float32),
            jax.ShapeDtypeStruct((E, De), jnp.float32),
        ],
    )(xs, ea, Wu1T, bu1, Wu2T, bu2, We1aT, We1bT, be1)


def _tc_prep2(h1s, e2, We2aT, We2bT, be2, P):
    E, D = h1s.shape
    De = e2.shape[1]
    Dh = We2aT.shape[1]
    BE = 2048

    def body(a_r, e_r, wa_r, wb_r, b_r, o_r):
        v = _relu(_dot(a_r[...], wa_r[...]) + _dot(e_r[...], wb_r[...])
                  + b_r[...])
        if P > Dh:
            v = jnp.concatenate([v, jnp.zeros((BE, P - Dh), jnp.float32)], 1)
        o_r[...] = v

    full = lambda shape: pl.BlockSpec(shape, lambda i: (0, 0))
    return pl.pallas_call(
        body,
        grid=(E // BE,),
        in_specs=[
            pl.BlockSpec((BE, D), lambda i: (i, 0)),
            pl.BlockSpec((BE, De), lambda i: (i, 0)),
            full((D, Dh)), full((De, Dh)), full((1, Dh)),
        ],
        out_specs=pl.BlockSpec((BE, P), lambda i: (i, 0)),
        out_shape=jax.ShapeDtypeStruct((E, P), jnp.float32),
    )(h1s, e2, We2aT, We2bT, be2)


def _tc_update(h0, mpos, mneg, WmT, bm):
    E, P = h0.shape
    Dh = WmT.shape[0]
    BE = 2048

    def body(h0_r, mp_r, mn_r, w_r, b_r, o_r):
        m = mp_r[...][:, :Dh] - mn_r[...][:, :Dh]
        v = _relu(h0_r[...][:, :Dh] + _dot(m, w_r[...]) + b_r[...])
        if P > Dh:
            v = jnp.concatenate([v, jnp.zeros((BE, P - Dh), jnp.float32)], 1)
        o_r[...] = v

    full = lambda shape: pl.BlockSpec(shape, lambda i: (0, 0))
    return pl.pallas_call(
        body,
        grid=(E // BE,),
        in_specs=[
            pl.BlockSpec((BE, P), lambda i: (i, 0)),
            pl.BlockSpec((BE, P), lambda i: (i, 0)),
            pl.BlockSpec((BE, P), lambda i: (i, 0)),
            full((Dh, Dh)), full((1, Dh)),
        ],
        out_specs=pl.BlockSpec((BE, P), lambda i: (i, 0)),
        out_shape=jax.ShapeDtypeStruct((E, P), jnp.float32),
    )(h0, mpos, mneg, WmT, bm)


def _tc_node_ln(xin, nm, WnaT, WnbT, bn, batchf, gamma, beta):
    N, D = xin.shape
    Do = WnaT.shape[1]
    Pm = nm.shape[1]
    G = _NUM_GRAPHS

    def body(x_r, nm_r, wa_r, wb_r, b_r, bat_r, g_r, be_r, o_r):
        n = _relu(_dot(x_r[...], wa_r[...]) + _dot(nm_r[...][:, :Do], wb_r[...])
                  + b_r[...])
        gids = lax.broadcasted_iota(jnp.int32, (N, G), 1).astype(jnp.float32)
        sel = (bat_r[...] == gids).astype(jnp.float32)
        cnt_nodes = jnp.sum(sel, axis=0, keepdims=True)
        cnt = jnp.maximum(cnt_nodes * float(Do), 1.0)
        rows = jnp.sum(n, axis=1, keepdims=True)
        gsum = lax.dot_general(sel, rows, (((0,), (0,)), ((), ())),
                               preferred_element_type=jnp.float32)
        mean_g = gsum / cnt.reshape(G, 1)
        mean_pn = _dot(sel, mean_g)
        xc = n - mean_pn
        rows2 = jnp.sum(xc * xc, axis=1, keepdims=True)
        vsum = lax.dot_general(sel, rows2, (((0,), (0,)), ((), ())),
                               preferred_element_type=jnp.float32)
        var_g = vsum / cnt.reshape(G, 1)
        var_pn = _dot(sel, var_g)
        ln = xc * lax.rsqrt(var_pn + 1e-5) * g_r[...] + be_r[...]
        o_r[...] = _elu(ln)

    full = lambda shape: pl.BlockSpec(shape, lambda i: (0, 0))
    return pl.pallas_call(
        body,
        grid=(1,),
        in_specs=[
            full((N, D)), full((N, Pm)),
            full((D, Do)), full((Do, Do)), full((1, Do)),
            full((N, 1)), full((1, Do)), full((1, Do)),
        ],
        out_specs=full((N, Do)),
        out_shape=jax.ShapeDtypeStruct((N, Do), jnp.float32),
    )(xin, nm, WnaT, WnbT, bn, batchf, gamma, beta)


def kernel(x, edge_index, edge_attr, batch, Wu1, bu1, Wu2, bu2, We1, be1,
           Wm1, bm1, Wn1, bn1, g1, b1, We2, be2, Wm2, bm2, Wn2, bn2, g2, b2):
    N, D = x.shape
    E = edge_index.shape[1]
    De = edge_attr.shape[1]
    Dh = Wm2.shape[0]

    src = edge_index[0].astype(jnp.int32)
    dst = edge_index[1].astype(jnp.int32)

    key = src * N + dst
    order = jnp.argsort(key)
    ks = key[order]
    starts = jnp.concatenate([jnp.ones((1,), jnp.int32),
                              (ks[1:] != ks[:-1]).astype(jnp.int32)])
    gid_sorted = jnp.cumsum(starts) - 1
    inv = jnp.zeros((E,), jnp.int32).at[order].set(gid_sorted)
    krev = dst * N + src
    pos = jnp.searchsorted(ks, krev)
    posc = jnp.minimum(pos, E - 1)
    found = (pos < E) & (ks[posc] == krev)

    PADD = 256
    H = N + PADD + E
    tpos = dst
    tneg = (N + PADD) + inv
    gsrc = src
    grev = jnp.where(found, (N + PADD) + gid_sorted[posc],
                     N + (jnp.arange(E, dtype=jnp.int32) % _L))

    as2d = lambda a: a.reshape(E // 128, 128)
    tpos2d, tneg2d = as2d(tpos), as2d(tneg)
    gsrc2d, grev2d = as2d(gsrc), as2d(grev)

    Wu1T, Wu2T = Wu1.T, Wu2.T
    We1aT, We1bT = We1[:, :D].T, We1[:, D:].T
    Wm1T = Wm1.T
    Wn1aT, Wn1bT = Wn1[:, :D].T, Wn1[:, D:].T
    We2aT, We2bT = We2[:, :D].T, We2[:, D:].T
    Wm2T = Wm2.T
    Wn2aT, Wn2bT = Wn2[:, :D].T, Wn2[:, D:].T
    row = lambda v: v.reshape(1, -1)
    batchf = batch.astype(jnp.float32).reshape(N, 1)

    xs = _sc_gather(x, gsrc2d)
    h0, e2 = _tc_prep1(xs, edge_attr, Wu1T, row(bu1), Wu2T, row(bu2),
                       We1aT, We1bT, row(be1))
    h = h0
    for _ in range(_T_STEPS):
        mpos, mneg = _sc_msg(h, tpos2d, tneg2d, gsrc2d, grev2d, H)
        h = _tc_update(h0, mpos, mneg, Wm1T, row(bm1))
    nm = _sc_nodemsg(h, tpos2d, N)
    h1 = _tc_node_ln(x, nm, Wn1aT, Wn1bT, row(bn1), batchf, row(g1), row(b1))

    h1s = _sc_gather(h1, gsrc2d)
    h02 = _tc_prep2(h1s, e2, We2aT, We2bT, row(be2), D)
    h = h02
    for _ in range(_T_STEPS):
        mpos, mneg = _sc_msg(h, tpos2d, tneg2d, gsrc2d, grev2d, H)
        h = _tc_update(h02, mpos, mneg, Wm2T, row(bm2))
    nm2 = _sc_nodemsg(h, tpos2d, N)
    h2 = _tc_node_ln(h1, nm2, Wn2aT, Wn2bT, row(bn2), batchf, row(g2), row(b2))
    return h2


# speedup vs baseline: 2.1362x; 1.2168x over previous
import functools

import jax
import jax.numpy as jnp
from jax import lax
from jax.experimental import pallas as pl
from jax.experimental.pallas import tpu as pltpu
from jax.experimental.pallas import tpu_sc as plsc

_NC = 2
_NS = 16
_L = 16

_T_STEPS = 3
_NUM_GRAPHS = 64


def _sc_gather(table, idx2d):
    R, D = table.shape
    nrow = idx2d.shape[0]
    E = nrow * 128
    rows_pt = nrow // (_NC * _NS)
    ept = rows_pt * 128
    mesh = plsc.VectorSubcoreMesh(core_axis_name="c", subcore_axis_name="s")

    @functools.partial(
        pl.kernel, mesh=mesh,
        out_type=jax.ShapeDtypeStruct((E, D), jnp.float32),
        scratch_types=[
            pltpu.VMEM((ept, D), jnp.float32),
            pltpu.VMEM((rows_pt, 128), jnp.int32),
            pltpu.SemaphoreType.DMA,
        ],
    )
    def k(table_hbm, idx_hbm, out_hbm, buf, idxv, sem):
        c = lax.axis_index("c")
        s = lax.axis_index("s")
        w = c * _NS + s
        rb = w * rows_pt
        pltpu.sync_copy(idx_hbm.at[pl.ds(rb, rows_pt)], idxv)
        for j in range(rows_pt):
            pltpu.async_copy(table_hbm.at[idxv.at[j]],
                             buf.at[pl.ds(j * 128, 128)], sem).wait()
        pltpu.sync_copy(buf, out_hbm.at[pl.ds(w * ept, ept)])

    return k(table, idx2d)


def _sc_msg(h, tpos2d, tneg2d, gsrc2d, grev2d, H):
    E, D = h.shape
    ept_s = E // _NS
    srows = ept_s // 128
    ept_g = E // (_NC * _NS)
    grows = ept_g // 128
    BUF = ept_g
    brows = BUF // 128
    zpt = H // _NS
    mesh = plsc.VectorSubcoreMesh(core_axis_name="c", subcore_axis_name="s")

    @functools.partial(
        pl.kernel, mesh=mesh,
        out_type=[jax.ShapeDtypeStruct((E, D), jnp.float32),
                  jax.ShapeDtypeStruct((E, D), jnp.float32)],
        scratch_types=[
            pltpu.VMEM_SHARED((H, D), jnp.float32),
            pltpu.VMEM((BUF, D), jnp.float32),
            pltpu.VMEM((_L, D), jnp.float32),
            pltpu.VMEM((srows, 128), jnp.int32),
            pltpu.VMEM((srows, 128), jnp.int32),
            pltpu.VMEM((grows, 128), jnp.int32),
            pltpu.VMEM((grows, 128), jnp.int32),
            pltpu.SemaphoreType.DMA,
        ],
    )
    def k(h_hbm, tpos_hbm, tneg_hbm, gsrc_hbm, grev_hbm, mpos_hbm, mneg_hbm,
          table, buf, zbuf, tposv, tnegv, gsrcv, grevv, sem):
        c = lax.axis_index("c")
        s = lax.axis_index("s")
        for i in range(_L):
            for q in range(D // _L):
                zbuf[i, pl.ds(q * _L, _L)] = jnp.zeros((_L,), jnp.float32)
        zcopies = [
            pltpu.async_copy(zbuf, table.at[pl.ds(s * zpt + i * _L, _L)], sem)
            for i in range(zpt // _L)
        ]
        for zc in zcopies:
            zc.wait()
        plsc.subcore_barrier()
        pltpu.sync_copy(tpos_hbm.at[pl.ds(s * srows, srows)], tposv)
        pltpu.sync_copy(tneg_hbm.at[pl.ds(s * srows, srows)], tnegv)
        for ch in range(ept_s // BUF):
            eb = s * ept_s + ch * BUF
            pltpu.sync_copy(h_hbm.at[pl.ds(eb, BUF)], buf)
            for j in range(brows):
                pltpu.sync_copy(buf.at[pl.ds(j * 128, 128)],
                                table.at[tposv.at[ch * brows + j]], add=True)
            for j in range(brows):
                pltpu.sync_copy(buf.at[pl.ds(j * 128, 128)],
                                table.at[tnegv.at[ch * brows + j]], add=True)
        plsc.subcore_barrier()
        w = c * _NS + s
        gb = w * ept_g
        pltpu.sync_copy(gsrc_hbm.at[pl.ds(w * grows, grows)], gsrcv)
        pltpu.sync_copy(grev_hbm.at[pl.ds(w * grows, grows)], grevv)
        for j in range(grows):
            pltpu.async_copy(table.at[gsrcv.at[j]],
                             buf.at[pl.ds(j * 128, 128)], sem).wait()
        pltpu.sync_copy(buf, mpos_hbm.at[pl.ds(gb, ept_g)])
        for j in range(grows):
            pltpu.async_copy(table.at[grevv.at[j]],
                             buf.at[pl.ds(j * 128, 128)], sem).wait()
        pltpu.sync_copy(buf, mneg_hbm.at[pl.ds(gb, ept_g)])

    return k(h, tpos2d, tneg2d, gsrc2d, grev2d)


def _sc_nodemsg(h, tpos2d, N):
    E, D = h.shape
    ept_s = E // _NS
    srows = ept_s // 128
    zpt = N // _NS
    npc = N // _NC
    nps = npc // _NS
    mesh = plsc.VectorSubcoreMesh(core_axis_name="c", subcore_axis_name="s")

    @functools.partial(
        pl.kernel, mesh=mesh,
        out_type=jax.ShapeDtypeStruct((N, D), jnp.float32),
        scratch_types=[
            pltpu.VMEM_SHARED((N, D), jnp.float32),
            pltpu.VMEM((ept_s, D), jnp.float32),
            pltpu.VMEM((_L, D), jnp.float32),
            pltpu.VMEM((srows, 128), jnp.int32),
        ],
    )
    def k(h_hbm, tpos_hbm, out_hbm, table, buf, zbuf, tposv):
        c = lax.axis_index("c")
        s = lax.axis_index("s")
        for i in range(_L):
            for q in range(D // _L):
                zbuf[i, pl.ds(q * _L, _L)] = jnp.zeros((_L,), jnp.float32)
        for i in range(zpt // _L):
            pltpu.sync_copy(zbuf, table.at[pl.ds(s * zpt + i * _L, _L)])
        plsc.subcore_barrier()
        eb = s * ept_s
        pltpu.sync_copy(h_hbm.at[pl.ds(eb, ept_s)], buf)
        pltpu.sync_copy(tpos_hbm.at[pl.ds(s * srows, srows)], tposv)
        for j in range(srows):
            pltpu.sync_copy(buf.at[pl.ds(j * 128, 128)],
                            table.at[tposv.at[j]], add=True)
        plsc.subcore_barrier()
        rb = c * npc + s * nps
        pltpu.sync_copy(table.at[pl.ds(rb, nps)], buf.at[pl.ds(0, nps)])
        pltpu.sync_copy(buf.at[pl.ds(0, nps)], out_hbm.at[pl.ds(rb, nps)])

    return k(h, tpos2d)


def _relu(v):
    return jnp.maximum(v, 0.0)


def _elu(v):
    return jnp.where(v > 0, v, jnp.exp(jnp.minimum(v, 0.0)) - 1.0)


def _dot(a, b):
    return jax.lax.dot_general(a, b, (((1,), (0,)), ((), ())),
                               precision=jax.lax.Precision.HIGHEST,
                               preferred_element_type=jnp.float32)


def _tc_prep1(xs, ea, Wu1T, bu1, Wu2T, bu2, We1aT, We1bT, be1):
    E, D = xs.shape
    De = ea.shape[1]
    BE = 2048

    def body(xs_r, ea_r, wu1_r, bu1_r, wu2_r, bu2_r, wa_r, wb_r, be_r,
             h0_r, e2_r):
        e1 = _elu(_dot(ea_r[...], wu1_r[...]) + bu1_r[...])
        e2_r[...] = _elu(_dot(e1, wu2_r[...]) + bu2_r[...])
        h0_r[...] = _relu(_dot(xs_r[...], wa_r[...]) + _dot(e1, wb_r[...])
                          + be_r[...])

    full = lambda shape: pl.BlockSpec(shape, lambda i: (0, 0))
    return pl.pallas_call(
        body,
        grid=(E // BE,),
        in_specs=[
            pl.BlockSpec((BE, D), lambda i: (i, 0)),
            pl.BlockSpec((BE, De), lambda i: (i, 0)),
            full((De, De)), full((1, De)), full((De, De)), full((1, De)),
            full((D, D)), full((De, D)), full((1, D)),
        ],
        out_specs=[
            pl.BlockSpec((BE, D), lambda i: (i, 0)),
            pl.BlockSpec((BE, De), lambda i: (i, 0)),
        ],
        out_shape=[
            jax.ShapeDtypeStruct((E, D), jnp.float32),
            jax.ShapeDtypeStruct((E, De), jnp.float32),
        ],
    )(xs, ea, Wu1T, bu1, Wu2T, bu2, We1aT, We1bT, be1)


def _tc_prep2(h1s, e2, We2aT, We2bT, be2, P):
    E, D = h1s.shape
    De = e2.shape[1]
    Dh = We2aT.shape[1]
    BE = 2048

    def body(a_r, e_r, wa_r, wb_r, b_r, o_r):
        v = _relu(_dot(a_r[...], wa_r[...]) + _dot(e_r[...], wb_r[...])
                  + b_r[...])
        if P > Dh:
            v = jnp.concatenate([v, jnp.zeros((BE, P - Dh), jnp.float32)], 1)
        o_r[...] = v

    full = lambda shape: pl.BlockSpec(shape, lambda i: (0, 0))
    return pl.pallas_call(
        body,
        grid=(E // BE,),
        in_specs=[
            pl.BlockSpec((BE, D), lambda i: (i, 0)),
            pl.BlockSpec((BE, De), lambda i: (i, 0)),
            full((D, Dh)), full((De, Dh)), full((1, Dh)),
        ],
        out_specs=pl.BlockSpec((BE, P), lambda i: (i, 0)),
        out_shape=jax.ShapeDtypeStruct((E, P), jnp.float32),
    )(h1s, e2, We2aT, We2bT, be2)


def _tc_update(h0, mpos, mneg, WmT, bm):
    E, P = h0.shape
    Dh = WmT.shape[0]
    BE = 2048

    def body(h0_r, mp_r, mn_r, w_r, b_r, o_r):
        m = mp_r[...][:, :Dh] - mn_r[...][:, :Dh]
        v = _relu(h0_r[...][:, :Dh] + _dot(m, w_r[...]) + b_r[...])
        if P > Dh:
            v = jnp.concatenate([v, jnp.zeros((BE, P - Dh), jnp.float32)], 1)
        o_r[...] = v

    full = lambda shape: pl.BlockSpec(shape, lambda i: (0, 0))
    return pl.pallas_call(
        body,
        grid=(E // BE,),
        in_specs=[
            pl.BlockSpec((BE, P), lambda i: (i, 0)),
            pl.BlockSpec((BE, P), lambda i: (i, 0)),
            pl.BlockSpec((BE, P), lambda i: (i, 0)),
            full((Dh, Dh)), full((1, Dh)),
        ],
        out_specs=pl.BlockSpec((BE, P), lambda i: (i, 0)),
        out_shape=jax.ShapeDtypeStruct((E, P), jnp.float32),
    )(h0, mpos, mneg, WmT, bm)


def _tc_node_ln(xin, nm, WnaT, WnbT, bn, batchf, gamma, beta):
    N, D = xin.shape
    Do = WnaT.shape[1]
    Pm = nm.shape[1]
    G = _NUM_GRAPHS

    def body(x_r, nm_r, wa_r, wb_r, b_r, bat_r, g_r, be_r, o_r):
        n = _relu(_dot(x_r[...], wa_r[...]) + _dot(nm_r[...][:, :Do], wb_r[...])
                  + b_r[...])
        gids = lax.broadcasted_iota(jnp.int32, (N, G), 1).astype(jnp.float32)
        sel = (bat_r[...] == gids).astype(jnp.float32)
        cnt_nodes = jnp.sum(sel, axis=0, keepdims=True)
        cnt = jnp.maximum(cnt_nodes * float(Do), 1.0)
        rows = jnp.sum(n, axis=1, keepdims=True)
        gsum = lax.dot_general(sel, rows, (((0,), (0,)), ((), ())),
                               preferred_element_type=jnp.float32)
        mean_g = gsum / cnt.reshape(G, 1)
        mean_pn = _dot(sel, mean_g)
        xc = n - mean_pn
        rows2 = jnp.sum(xc * xc, axis=1, keepdims=True)
        vsum = lax.dot_general(sel, rows2, (((0,), (0,)), ((), ())),
                               preferred_element_type=jnp.float32)
        var_g = vsum / cnt.reshape(G, 1)
        var_pn = _dot(sel, var_g)
        ln = xc * lax.rsqrt(var_pn + 1e-5) * g_r[...] + be_r[...]
        o_r[...] = _elu(ln)

    full = lambda shape: pl.BlockSpec(shape, lambda i: (0, 0))
    return pl.pallas_call(
        body,
        grid=(1,),
        in_specs=[
            full((N, D)), full((N, Pm)),
            full((D, Do)), full((Do, Do)), full((1, Do)),
            full((N, 1)), full((1, Do)), full((1, Do)),
        ],
        out_specs=full((N, Do)),
        out_shape=jax.ShapeDtypeStruct((N, Do), jnp.float32),
    )(xin, nm, WnaT, WnbT, bn, batchf, gamma, beta)


def kernel(x, edge_index, edge_attr, batch, Wu1, bu1, Wu2, bu2, We1, be1,
           Wm1, bm1, Wn1, bn1, g1, b1, We2, be2, Wm2, bm2, Wn2, bn2, g2, b2):
    N, D = x.shape
    E = edge_index.shape[1]
    De = edge_attr.shape[1]
    Dh = Wm2.shape[0]

    src = edge_index[0].astype(jnp.int32)
    dst = edge_index[1].astype(jnp.int32)

    PADD = 256
    H = N + PADD + E
    key = src * N + dst
    krev = dst * N + src
    E2 = 2 * E
    comb = jnp.concatenate([key * 2, krev * 2 + 1])
    csort, cidx = lax.sort_key_val(comb, lax.iota(jnp.int32, E2))
    ckey = csort >> 1
    ctag = csort & 1
    is_new = jnp.concatenate([jnp.ones((1,), jnp.bool_), ckey[1:] != ckey[:-1]])
    nk0 = is_new & (ctag == 0)
    grun = jnp.cumsum(nk0.astype(jnp.int32)) - 1
    pos2 = lax.iota(jnp.int32, E2)
    zz = lax.cummax(jnp.where(is_new, pos2 * 2 + ctag, -1))
    found_run = (zz & 1) == 0
    orig = cidx - E
    eidx1 = jnp.where(ctag == 1, orig, E)
    val1 = jnp.where(found_run, (N + PADD) + grun, N + (orig & (_L - 1)))
    grev = jnp.zeros((E + 1,), jnp.int32).at[eidx1].set(val1)[:E]
    eidx0 = jnp.where(ctag == 0, cidx, E)
    inv = jnp.zeros((E + 1,), jnp.int32).at[eidx0].set(grun)[:E]

    tpos = dst
    tneg = (N + PADD) + inv
    gsrc = src

    as2d = lambda a: a.reshape(E // 128, 128)
    tpos2d, tneg2d = as2d(tpos), as2d(tneg)
    gsrc2d, grev2d = as2d(gsrc), as2d(grev)

    Wu1T, Wu2T = Wu1.T, Wu2.T
    We1aT, We1bT = We1[:, :D].T, We1[:, D:].T
    Wm1T = Wm1.T
    Wn1aT, Wn1bT = Wn1[:, :D].T, Wn1[:, D:].T
    We2aT, We2bT = We2[:, :D].T, We2[:, D:].T
    Wm2T = Wm2.T
    Wn2aT, Wn2bT = Wn2[:, :D].T, Wn2[:, D:].T
    row = lambda v: v.reshape(1, -1)
    batchf = batch.astype(jnp.float32).reshape(N, 1)

    xs = _sc_gather(x, gsrc2d)
    h0, e2 = _tc_prep1(xs, edge_attr, Wu1T, row(bu1), Wu2T, row(bu2),
                       We1aT, We1bT, row(be1))
    h = h0
    for _ in range(_T_STEPS):
        mpos, mneg = _sc_msg(h, tpos2d, tneg2d, gsrc2d, grev2d, H)
        h = _tc_update(h0, mpos, mneg, Wm1T, row(bm1))
    nm = _sc_nodemsg(h, tpos2d, N)
    h1 = _tc_node_ln(x, nm, Wn1aT, Wn1bT, row(bn1), batchf, row(g1), row(b1))

    h1s = _sc_gather(h1, gsrc2d)
    h02 = _tc_prep2(h1s, e2, We2aT, We2bT, row(be2), D)
    h = h02
    for _ in range(_T_STEPS):
        mpos, mneg = _sc_msg(h, tpos2d, tneg2d, gsrc2d, grev2d, H)
        h = _tc_update(h02, mpos, mneg, Wm2T, row(bm2))
    nm2 = _sc_nodemsg(h, tpos2d, N)
    h2 = _tc_node_ln(h1, nm2, Wn2aT, Wn2bT, row(bn2), batchf, row(g2), row(b2))
    return h2


# speedup vs baseline: 2.3623x; 1.1058x over previous
import functools

import jax
import jax.numpy as jnp
from jax import lax
from jax.experimental import pallas as pl
from jax.experimental.pallas import tpu as pltpu
from jax.experimental.pallas import tpu_sc as plsc

_NC = 2
_NS = 16
_L = 16

_T_STEPS = 3
_NUM_GRAPHS = 64


def _sc_gather(table, idx2d):
    R, D = table.shape
    nrow = idx2d.shape[0]
    E = nrow * 128
    rows_pt = nrow // (_NC * _NS)
    ept = rows_pt * 128
    mesh = plsc.VectorSubcoreMesh(core_axis_name="c", subcore_axis_name="s")

    @functools.partial(
        pl.kernel, mesh=mesh,
        out_type=jax.ShapeDtypeStruct((E, D), jnp.float32),
        scratch_types=[
            pltpu.VMEM((ept, D), jnp.float32),
            pltpu.VMEM((rows_pt, 128), jnp.int32),
            pltpu.SemaphoreType.DMA,
        ],
    )
    def k(table_hbm, idx_hbm, out_hbm, buf, idxv, sem):
        c = lax.axis_index("c")
        s = lax.axis_index("s")
        w = c * _NS + s
        rb = w * rows_pt
        pltpu.sync_copy(idx_hbm.at[pl.ds(rb, rows_pt)], idxv)
        for j in range(rows_pt):
            pltpu.async_copy(table_hbm.at[idxv.at[j]],
                             buf.at[pl.ds(j * 128, 128)], sem).wait()
        pltpu.sync_copy(buf, out_hbm.at[pl.ds(w * ept, ept)])

    return k(table, idx2d)


def _sc_msg(h, tpos2d, tneg2d, gsrc2d, grev2d, H):
    E, D = h.shape
    ept_s = E // _NS
    srows = ept_s // 128
    ept_g = E // (_NC * _NS)
    grows = ept_g // 128
    zpt = H // _NS
    mesh = plsc.VectorSubcoreMesh(core_axis_name="c", subcore_axis_name="s")

    @functools.partial(
        pl.kernel, mesh=mesh,
        out_type=[jax.ShapeDtypeStruct((E, D), jnp.float32),
                  jax.ShapeDtypeStruct((E, D), jnp.float32)],
        scratch_types=[
            pltpu.VMEM_SHARED((H, D), jnp.float32),
            pltpu.VMEM((128, D), jnp.float32),
            pltpu.VMEM((128, D), jnp.float32),
            pltpu.VMEM((_L, D), jnp.float32),
            pltpu.VMEM((srows, 128), jnp.int32),
            pltpu.VMEM((srows, 128), jnp.int32),
            pltpu.VMEM((grows, 128), jnp.int32),
            pltpu.VMEM((grows, 128), jnp.int32),
            pltpu.SemaphoreType.DMA,
            pltpu.SemaphoreType.DMA,
            pltpu.SemaphoreType.DMA,
            pltpu.SemaphoreType.DMA,
        ],
    )
    def k(h_hbm, tpos_hbm, tneg_hbm, gsrc_hbm, grev_hbm, mpos_hbm, mneg_hbm,
          table, buf0, buf1, zbuf, tposv, tnegv, gsrcv, grevv,
          semz, semh, semg, semw):
        c = lax.axis_index("c")
        s = lax.axis_index("s")
        w = c * _NS + s
        bufs = (buf0, buf1)
        for i in range(_L):
            for q in range(D // _L):
                zbuf[i, pl.ds(q * _L, _L)] = jnp.zeros((_L,), jnp.float32)
        zcopies = [
            pltpu.async_copy(zbuf, table.at[pl.ds(s * zpt + i * _L, _L)], semz)
            for i in range(zpt // _L)
        ]
        icopies = [
            pltpu.async_copy(tpos_hbm.at[pl.ds(s * srows, srows)], tposv, semh),
            pltpu.async_copy(tneg_hbm.at[pl.ds(s * srows, srows)], tnegv, semh),
            pltpu.async_copy(gsrc_hbm.at[pl.ds(w * grows, grows)], gsrcv, semh),
            pltpu.async_copy(grev_hbm.at[pl.ds(w * grows, grows)], grevv, semh),
        ]
        eb = s * ept_s
        first = pltpu.async_copy(h_hbm.at[pl.ds(eb, 128)], buf0, semh)
        for zc in zcopies:
            zc.wait()
        for ic in icopies:
            ic.wait()
        first.wait()
        plsc.subcore_barrier()
        for ch in range(srows):
            if ch + 1 < srows:
                nxt = pltpu.async_copy(
                    h_hbm.at[pl.ds(eb + (ch + 1) * 128, 128)],
                    bufs[(ch + 1) % 2], semh)
            b = bufs[ch % 2]
            pltpu.sync_copy(b, table.at[tposv.at[ch]], add=True)
            pltpu.sync_copy(b, table.at[tnegv.at[ch]], add=True)
            if ch + 1 < srows:
                nxt.wait()
        plsc.subcore_barrier()
        gb = w * ept_g
        g0 = pltpu.async_copy(table.at[gsrcv.at[0]], buf0, semg)
        g1 = pltpu.async_copy(table.at[gsrcv.at[1]], buf1, semg)
        g0.wait()
        w0 = pltpu.async_copy(buf0, mpos_hbm.at[pl.ds(gb, 128)], semw)
        g1.wait()
        w1 = pltpu.async_copy(buf1, mpos_hbm.at[pl.ds(gb + 128, 128)], semw)
        w0.wait()
        g2 = pltpu.async_copy(table.at[grevv.at[0]], buf0, semg)
        w1.wait()
        g3 = pltpu.async_copy(table.at[grevv.at[1]], buf1, semg)
        g2.wait()
        w2 = pltpu.async_copy(buf0, mneg_hbm.at[pl.ds(gb, 128)], semw)
        g3.wait()
        w3 = pltpu.async_copy(buf1, mneg_hbm.at[pl.ds(gb + 128, 128)], semw)
        w2.wait()
        w3.wait()

    return k(h, tpos2d, tneg2d, gsrc2d, grev2d)


def _sc_nodemsg(h, tpos2d, N):
    E, D = h.shape
    ept_s = E // _NS
    srows = ept_s // 128
    zpt = N // _NS
    npc = N // _NC
    nps = npc // _NS
    mesh = plsc.VectorSubcoreMesh(core_axis_name="c", subcore_axis_name="s")

    @functools.partial(
        pl.kernel, mesh=mesh,
        out_type=jax.ShapeDtypeStruct((N, D), jnp.float32),
        scratch_types=[
            pltpu.VMEM_SHARED((N, D), jnp.float32),
            pltpu.VMEM((ept_s, D), jnp.float32),
            pltpu.VMEM((_L, D), jnp.float32),
            pltpu.VMEM((srows, 128), jnp.int32),
        ],
    )
    def k(h_hbm, tpos_hbm, out_hbm, table, buf, zbuf, tposv):
        c = lax.axis_index("c")
        s = lax.axis_index("s")
        for i in range(_L):
            for q in range(D // _L):
                zbuf[i, pl.ds(q * _L, _L)] = jnp.zeros((_L,), jnp.float32)
        for i in range(zpt // _L):
            pltpu.sync_copy(zbuf, table.at[pl.ds(s * zpt + i * _L, _L)])
        plsc.subcore_barrier()
        eb = s * ept_s
        pltpu.sync_copy(h_hbm.at[pl.ds(eb, ept_s)], buf)
        pltpu.sync_copy(tpos_hbm.at[pl.ds(s * srows, srows)], tposv)
        for j in range(srows):
            pltpu.sync_copy(buf.at[pl.ds(j * 128, 128)],
                            table.at[tposv.at[j]], add=True)
        plsc.subcore_barrier()
        rb = c * npc + s * nps
        pltpu.sync_copy(table.at[pl.ds(rb, nps)], buf.at[pl.ds(0, nps)])
        pltpu.sync_copy(buf.at[pl.ds(0, nps)], out_hbm.at[pl.ds(rb, nps)])

    return k(h, tpos2d)


def _relu(v):
    return jnp.maximum(v, 0.0)


def _elu(v):
    return jnp.where(v > 0, v, jnp.exp(jnp.minimum(v, 0.0)) - 1.0)


def _dot(a, b):
    return jax.lax.dot_general(a, b, (((1,), (0,)), ((), ())),
                               precision=jax.lax.Precision.HIGHEST,
                               preferred_element_type=jnp.float32)


def _tc_prep1(xs, ea, Wu1T, bu1, Wu2T, bu2, We1aT, We1bT, be1):
    E, D = xs.shape
    De = ea.shape[1]
    BE = 2048

    def body(xs_r, ea_r, wu1_r, bu1_r, wu2_r, bu2_r, wa_r, wb_r, be_r,
             h0_r, e2_r):
        e1 = _elu(_dot(ea_r[...], wu1_r[...]) + bu1_r[...])
        e2_r[...] = _elu(_dot(e1, wu2_r[...]) + bu2_r[...])
        h0_r[...] = _relu(_dot(xs_r[...], wa_r[...]) + _dot(e1, wb_r[...])
                          + be_r[...])

    full = lambda shape: pl.BlockSpec(shape, lambda i: (0, 0))
    return pl.pallas_call(
        body,
        grid=(E // BE,),
        in_specs=[
            pl.BlockSpec((BE, D), lambda i: (i, 0)),
            pl.BlockSpec((BE, De), lambda i: (i, 0)),
            full((De, De)), full((1, De)), full((De, De)), full((1, De)),
            full((D, D)), full((De, D)), full((1, D)),
        ],
        out_specs=[
            pl.BlockSpec((BE, D), lambda i: (i, 0)),
            pl.BlockSpec((BE, De), lambda i: (i, 0)),
        ],
        out_shape=[
            jax.ShapeDtypeStruct((E, D), jnp.float32),
            jax.ShapeDtypeStruct((E, De), jnp.float32),
        ],
    )(xs, ea, Wu1T, bu1, Wu2T, bu2, We1aT, We1bT, be1)


def _tc_prep2(h1s, e2, We2aT, We2bT, be2, P):
    E, D = h1s.shape
    De = e2.shape[1]
    Dh = We2aT.shape[1]
    BE = 2048

    def body(a_r, e_r, wa_r, wb_r, b_r, o_r):
        v = _relu(_dot(a_r[...], wa_r[...]) + _dot(e_r[...], wb_r[...])
                  + b_r[...])
        if P > Dh:
            v = jnp.concatenate([v, jnp.zeros((BE, P - Dh), jnp.float32)], 1)
        o_r[...] = v

    full = lambda shape: pl.BlockSpec(shape, lambda i: (0, 0))
    return pl.pallas_call(
        body,
        grid=(E // BE,),
        in_specs=[
            pl.BlockSpec((BE, D), lambda i: (i, 0)),
            pl.BlockSpec((BE, De), lambda i: (i, 0)),
            full((D, Dh)), full((De, Dh)), full((1, Dh)),
        ],
        out_specs=pl.BlockSpec((BE, P), lambda i: (i, 0)),
        out_shape=jax.ShapeDtypeStruct((E, P), jnp.float32),
    )(h1s, e2, We2aT, We2bT, be2)


def _tc_update(h0, mpos, mneg, WmT, bm):
    E, P = h0.shape
    Dh = WmT.shape[0]
    BE = 2048

    def body(h0_r, mp_r, mn_r, w_r, b_r, o_r):
        m = mp_r[...][:, :Dh] - mn_r[...][:, :Dh]
        v = _relu(h0_r[...][:, :Dh] + _dot(m, w_r[...]) + b_r[...])
        if P > Dh:
            v = jnp.concatenate([v, jnp.zeros((BE, P - Dh), jnp.float32)], 1)
        o_r[...] = v

    full = lambda shape: pl.BlockSpec(shape, lambda i: (0, 0))
    return pl.pallas_call(
        body,
        grid=(E // BE,),
        in_specs=[
            pl.BlockSpec((BE, P), lambda i: (i, 0)),
            pl.BlockSpec((BE, P), lambda i: (i, 0)),
            pl.BlockSpec((BE, P), lambda i: (i, 0)),
            full((Dh, Dh)), full((1, Dh)),
        ],
        out_specs=pl.BlockSpec((BE, P), lambda i: (i, 0)),
        out_shape=jax.ShapeDtypeStruct((E, P), jnp.float32),
    )(h0, mpos, mneg, WmT, bm)


def _tc_node_ln(xin, nm, WnaT, WnbT, bn, batchf, gamma, beta):
    N, D = xin.shape
    Do = WnaT.shape[1]
    Pm = nm.shape[1]
    G = _NUM_GRAPHS

    def body(x_r, nm_r, wa_r, wb_r, b_r, bat_r, g_r, be_r, o_r):
        n = _relu(_dot(x_r[...], wa_r[...]) + _dot(nm_r[...][:, :Do], wb_r[...])
                  + b_r[...])
        gids = lax.broadcasted_iota(jnp.int32, (N, G), 1).astype(jnp.float32)
        sel = (bat_r[...] == gids).astype(jnp.float32)
        cnt_nodes = jnp.sum(sel, axis=0, keepdims=True)
        cnt = jnp.maximum(cnt_nodes * float(Do), 1.0)
        rows = jnp.sum(n, axis=1, keepdims=True)
        gsum = lax.dot_general(sel, rows, (((0,), (0,)), ((), ())),
                               preferred_element_type=jnp.float32)
        mean_g = gsum / cnt.reshape(G, 1)
        mean_pn = _dot(sel, mean_g)
        xc = n - mean_pn
        rows2 = jnp.sum(xc * xc, axis=1, keepdims=True)
        vsum = lax.dot_general(sel, rows2, (((0,), (0,)), ((), ())),
                               preferred_element_type=jnp.float32)
        var_g = vsum / cnt.reshape(G, 1)
        var_pn = _dot(sel, var_g)
        ln = xc * lax.rsqrt(var_pn + 1e-5) * g_r[...] + be_r[...]
        o_r[...] = _elu(ln)

    full = lambda shape: pl.BlockSpec(shape, lambda i: (0, 0))
    return pl.pallas_call(
        body,
        grid=(1,),
        in_specs=[
            full((N, D)), full((N, Pm)),
            full((D, Do)), full((Do, Do)), full((1, Do)),
            full((N, 1)), full((1, Do)), full((1, Do)),
        ],
        out_specs=full((N, Do)),
        out_shape=jax.ShapeDtypeStruct((N, Do), jnp.float32),
    )(xin, nm, WnaT, WnbT, bn, batchf, gamma, beta)


def kernel(x, edge_index, edge_attr, batch, Wu1, bu1, Wu2, bu2, We1, be1,
           Wm1, bm1, Wn1, bn1, g1, b1, We2, be2, Wm2, bm2, Wn2, bn2, g2, b2):
    N, D = x.shape
    E = edge_index.shape[1]
    De = edge_attr.shape[1]
    Dh = Wm2.shape[0]

    src = edge_index[0].astype(jnp.int32)
    dst = edge_index[1].astype(jnp.int32)

    PADD = 256
    H = N + PADD + E
    key = src * N + dst
    krev = dst * N + src
    E2 = 2 * E
    comb = jnp.concatenate([key * 2, krev * 2 + 1])
    csort, cidx = lax.sort_key_val(comb, lax.iota(jnp.int32, E2))
    ckey = csort >> 1
    ctag = csort & 1
    is_new = jnp.concatenate([jnp.ones((1,), jnp.bool_), ckey[1:] != ckey[:-1]])
    nk0 = is_new & (ctag == 0)
    grun = jnp.cumsum(nk0.astype(jnp.int32)) - 1
    pos2 = lax.iota(jnp.int32, E2)
    zz = lax.cummax(jnp.where(is_new, pos2 * 2 + ctag, -1))
    found_run = (zz & 1) == 0
    orig = cidx - E
    eidx1 = jnp.where(ctag == 1, orig, E)
    val1 = jnp.where(found_run, (N + PADD) + grun, N + (orig & (_L - 1)))
    grev = jnp.zeros((E + 1,), jnp.int32).at[eidx1].set(val1)[:E]
    eidx0 = jnp.where(ctag == 0, cidx, E)
    inv = jnp.zeros((E + 1,), jnp.int32).at[eidx0].set(grun)[:E]

    tpos = dst
    tneg = (N + PADD) + inv
    gsrc = src

    as2d = lambda a: a.reshape(E // 128, 128)
    tpos2d, tneg2d = as2d(tpos), as2d(tneg)
    gsrc2d, grev2d = as2d(gsrc), as2d(grev)

    Wu1T, Wu2T = Wu1.T, Wu2.T
    We1aT, We1bT = We1[:, :D].T, We1[:, D:].T
    Wm1T = Wm1.T
    Wn1aT, Wn1bT = Wn1[:, :D].T, Wn1[:, D:].T
    We2aT, We2bT = We2[:, :D].T, We2[:, D:].T
    Wm2T = Wm2.T
    Wn2aT, Wn2bT = Wn2[:, :D].T, Wn2[:, D:].T
    row = lambda v: v.reshape(1, -1)
    batchf = batch.astype(jnp.float32).reshape(N, 1)

    xs = _sc_gather(x, gsrc2d)
    h0, e2 = _tc_prep1(xs, edge_attr, Wu1T, row(bu1), Wu2T, row(bu2),
                       We1aT, We1bT, row(be1))
    h = h0
    for _ in range(_T_STEPS):
        mpos, mneg = _sc_msg(h, tpos2d, tneg2d, gsrc2d, grev2d, H)
        h = _tc_update(h0, mpos, mneg, Wm1T, row(bm1))
    nm = _sc_nodemsg(h, tpos2d, N)
    h1 = _tc_node_ln(x, nm, Wn1aT, Wn1bT, row(bn1), batchf, row(g1), row(b1))

    h1s = _sc_gather(h1, gsrc2d)
    h02 = _tc_prep2(h1s, e2, We2aT, We2bT, row(be2), D)
    h = h02
    for _ in range(_T_STEPS):
        mpos, mneg = _sc_msg(h, tpos2d, tneg2d, gsrc2d, grev2d, H)
        h = _tc_update(h02, mpos, mneg, Wm2T, row(bm2))
    nm2 = _sc_nodemsg(h, tpos2d, N)
    h2 = _tc_node_ln(h1, nm2, Wn2aT, Wn2bT, row(bn2), batchf, row(g2), row(b2))
    return h2


# speedup vs baseline: 2.3778x; 1.0066x over previous
import functools

import jax
import jax.numpy as jnp
from jax import lax
from jax.experimental import pallas as pl
from jax.experimental.pallas import tpu as pltpu
from jax.experimental.pallas import tpu_sc as plsc

_NC = 2
_NS = 16
_L = 16

_T_STEPS = 3
_NUM_GRAPHS = 64


def _sc_gather(table, idx2d):
    R, D = table.shape
    nrow = idx2d.shape[0]
    E = nrow * 128
    rows_pt = nrow // (_NC * _NS)
    ept = rows_pt * 128
    mesh = plsc.VectorSubcoreMesh(core_axis_name="c", subcore_axis_name="s")

    @functools.partial(
        pl.kernel, mesh=mesh,
        out_type=jax.ShapeDtypeStruct((E, D), jnp.float32),
        scratch_types=[
            pltpu.VMEM((ept, D), jnp.float32),
            pltpu.VMEM((rows_pt, 128), jnp.int32),
            pltpu.SemaphoreType.DMA,
            pltpu.SemaphoreType.DMA,
        ],
    )
    def k(table_hbm, idx_hbm, out_hbm, buf, idxv, semg, semw):
        c = lax.axis_index("c")
        s = lax.axis_index("s")
        w = c * _NS + s
        rb = w * rows_pt
        pltpu.sync_copy(idx_hbm.at[pl.ds(rb, rows_pt)], idxv)
        gs = [pltpu.async_copy(table_hbm.at[idxv.at[j]],
                               buf.at[pl.ds(j * 128, 128)], semg)
              for j in range(rows_pt)]
        ws = []
        for j in range(rows_pt):
            gs[j].wait()
            ws.append(pltpu.async_copy(
                buf.at[pl.ds(j * 128, 128)],
                out_hbm.at[pl.ds(w * ept + j * 128, 128)], semw))
        for wc in ws:
            wc.wait()

    return k(table, idx2d)


def _sc_msg(h, tpos2d, tneg2d, gsrc2d, grev2d, H):
    E, D = h.shape
    ept_s = E // _NS
    srows = ept_s // 128
    ept_g = E // (_NC * _NS)
    grows = ept_g // 128
    zpt = H // _NS
    mesh = plsc.VectorSubcoreMesh(core_axis_name="c", subcore_axis_name="s")

    @functools.partial(
        pl.kernel, mesh=mesh,
        out_type=[jax.ShapeDtypeStruct((E, D), jnp.float32),
                  jax.ShapeDtypeStruct((E, D), jnp.float32)],
        scratch_types=[
            pltpu.VMEM_SHARED((H, D), jnp.float32),
            pltpu.VMEM((128, D), jnp.float32),
            pltpu.VMEM((128, D), jnp.float32),
            pltpu.VMEM((_L, D), jnp.float32),
            pltpu.VMEM((srows, 128), jnp.int32),
            pltpu.VMEM((srows, 128), jnp.int32),
            pltpu.VMEM((grows, 128), jnp.int32),
            pltpu.VMEM((grows, 128), jnp.int32),
            pltpu.SemaphoreType.DMA,
            pltpu.SemaphoreType.DMA,
            pltpu.SemaphoreType.DMA,
            pltpu.SemaphoreType.DMA,
        ],
    )
    def k(h_hbm, tpos_hbm, tneg_hbm, gsrc_hbm, grev_hbm, mpos_hbm, mneg_hbm,
          table, buf0, buf1, zbuf, tposv, tnegv, gsrcv, grevv,
          semz, semh, semg, semw):
        c = lax.axis_index("c")
        s = lax.axis_index("s")
        w = c * _NS + s
        bufs = (buf0, buf1)
        for i in range(_L):
            for q in range(D // _L):
                zbuf[i, pl.ds(q * _L, _L)] = jnp.zeros((_L,), jnp.float32)
        zcopies = [
            pltpu.async_copy(zbuf, table.at[pl.ds(s * zpt + i * _L, _L)], semz)
            for i in range(zpt // _L)
        ]
        icopies = [
            pltpu.async_copy(tpos_hbm.at[pl.ds(s * srows, srows)], tposv, semh),
            pltpu.async_copy(tneg_hbm.at[pl.ds(s * srows, srows)], tnegv, semh),
            pltpu.async_copy(gsrc_hbm.at[pl.ds(w * grows, grows)], gsrcv, semh),
            pltpu.async_copy(grev_hbm.at[pl.ds(w * grows, grows)], grevv, semh),
        ]
        eb = s * ept_s
        first = pltpu.async_copy(h_hbm.at[pl.ds(eb, 128)], buf0, semh)
        for zc in zcopies:
            zc.wait()
        for ic in icopies:
            ic.wait()
        first.wait()
        plsc.subcore_barrier()
        for ch in range(srows):
            if ch + 1 < srows:
                nxt = pltpu.async_copy(
                    h_hbm.at[pl.ds(eb + (ch + 1) * 128, 128)],
                    bufs[(ch + 1) % 2], semh)
            b = bufs[ch % 2]
            pltpu.sync_copy(b, table.at[tposv.at[ch]], add=True)
            pltpu.sync_copy(b, table.at[tnegv.at[ch]], add=True)
            if ch + 1 < srows:
                nxt.wait()
        plsc.subcore_barrier()
        gb = w * ept_g
        g0 = pltpu.async_copy(table.at[gsrcv.at[0]], buf0, semg)
        g1 = pltpu.async_copy(table.at[gsrcv.at[1]], buf1, semg)
        g0.wait()
        w0 = pltpu.async_copy(buf0, mpos_hbm.at[pl.ds(gb, 128)], semw)
        g1.wait()
        w1 = pltpu.async_copy(buf1, mpos_hbm.at[pl.ds(gb + 128, 128)], semw)
        w0.wait()
        g2 = pltpu.async_copy(table.at[grevv.at[0]], buf0, semg)
        w1.wait()
        g3 = pltpu.async_copy(table.at[grevv.at[1]], buf1, semg)
        g2.wait()
        w2 = pltpu.async_copy(buf0, mneg_hbm.at[pl.ds(gb, 128)], semw)
        g3.wait()
        w3 = pltpu.async_copy(buf1, mneg_hbm.at[pl.ds(gb + 128, 128)], semw)
        w2.wait()
        w3.wait()

    return k(h, tpos2d, tneg2d, gsrc2d, grev2d)


def _sc_nodemsg(h, tpos2d, N):
    E, D = h.shape
    ept_s = E // _NS
    srows = ept_s // 128
    zpt = N // _NS
    npc = N // _NC
    nps = npc // _NS
    mesh = plsc.VectorSubcoreMesh(core_axis_name="c", subcore_axis_name="s")

    @functools.partial(
        pl.kernel, mesh=mesh,
        out_type=jax.ShapeDtypeStruct((N, D), jnp.float32),
        scratch_types=[
            pltpu.VMEM_SHARED((N, D), jnp.float32),
            pltpu.VMEM((128, D), jnp.float32),
            pltpu.VMEM((128, D), jnp.float32),
            pltpu.VMEM((_L, D), jnp.float32),
            pltpu.VMEM((srows, 128), jnp.int32),
            pltpu.SemaphoreType.DMA,
            pltpu.SemaphoreType.DMA,
        ],
    )
    def k(h_hbm, tpos_hbm, out_hbm, table, buf0, buf1, zbuf, tposv,
          semz, semh):
        c = lax.axis_index("c")
        s = lax.axis_index("s")
        bufs = (buf0, buf1)
        for i in range(_L):
            for q in range(D // _L):
                zbuf[i, pl.ds(q * _L, _L)] = jnp.zeros((_L,), jnp.float32)
        zcopies = [
            pltpu.async_copy(zbuf, table.at[pl.ds(s * zpt + i * _L, _L)], semz)
            for i in range(zpt // _L)
        ]
        ic = pltpu.async_copy(tpos_hbm.at[pl.ds(s * srows, srows)], tposv, semh)
        eb = s * ept_s
        first = pltpu.async_copy(h_hbm.at[pl.ds(eb, 128)], buf0, semh)
        for zc in zcopies:
            zc.wait()
        ic.wait()
        first.wait()
        plsc.subcore_barrier()
        for ch in range(srows):
            if ch + 1 < srows:
                nxt = pltpu.async_copy(
                    h_hbm.at[pl.ds(eb + (ch + 1) * 128, 128)],
                    bufs[(ch + 1) % 2], semh)
            pltpu.sync_copy(bufs[ch % 2], table.at[tposv.at[ch]], add=True)
            if ch + 1 < srows:
                nxt.wait()
        plsc.subcore_barrier()
        rb = c * npc + s * nps
        pltpu.sync_copy(table.at[pl.ds(rb, nps)], buf0.at[pl.ds(0, nps)])
        pltpu.sync_copy(buf0.at[pl.ds(0, nps)], out_hbm.at[pl.ds(rb, nps)])

    return k(h, tpos2d)


def _relu(v):
    return jnp.maximum(v, 0.0)


def _elu(v):
    return jnp.where(v > 0, v, jnp.exp(jnp.minimum(v, 0.0)) - 1.0)


def _dot(a, b):
    return jax.lax.dot_general(a, b, (((1,), (0,)), ((), ())),
                               precision=jax.lax.Precision.HIGHEST,
                               preferred_element_type=jnp.float32)


def _tc_prep1(xs, ea, Wu1T, bu1, Wu2T, bu2, We1aT, We1bT, be1):
    E, D = xs.shape
    De = ea.shape[1]
    BE = 2048

    def body(xs_r, ea_r, wu1_r, bu1_r, wu2_r, bu2_r, wa_r, wb_r, be_r,
             h0_r, e2_r):
        e1 = _elu(_dot(ea_r[...], wu1_r[...]) + bu1_r[...])
        e2_r[...] = _elu(_dot(e1, wu2_r[...]) + bu2_r[...])
        h0_r[...] = _relu(_dot(xs_r[...], wa_r[...]) + _dot(e1, wb_r[...])
                          + be_r[...])

    full = lambda shape: pl.BlockSpec(shape, lambda i: (0, 0))
    return pl.pallas_call(
        body,
        grid=(E // BE,),
        in_specs=[
            pl.BlockSpec((BE, D), lambda i: (i, 0)),
            pl.BlockSpec((BE, De), lambda i: (i, 0)),
            full((De, De)), full((1, De)), full((De, De)), full((1, De)),
            full((D, D)), full((De, D)), full((1, D)),
        ],
        out_specs=[
            pl.BlockSpec((BE, D), lambda i: (i, 0)),
            pl.BlockSpec((BE, De), lambda i: (i, 0)),
        ],
        out_shape=[
            jax.ShapeDtypeStruct((E, D), jnp.float32),
            jax.ShapeDtypeStruct((E, De), jnp.float32),
        ],
    )(xs, ea, Wu1T, bu1, Wu2T, bu2, We1aT, We1bT, be1)


def _tc_prep2(h1s, e2, We2aT, We2bT, be2, P):
    E, D = h1s.shape
    De = e2.shape[1]
    Dh = We2aT.shape[1]
    BE = 2048

    def body(a_r, e_r, wa_r, wb_r, b_r, o_r):
        v = _relu(_dot(a_r[...], wa_r[...]) + _dot(e_r[...], wb_r[...])
                  + b_r[...])
        if P > Dh:
            v = jnp.concatenate([v, jnp.zeros((BE, P - Dh), jnp.float32)], 1)
        o_r[...] = v

    full = lambda shape: pl.BlockSpec(shape, lambda i: (0, 0))
    return pl.pallas_call(
        body,
        grid=(E // BE,),
        in_specs=[
            pl.BlockSpec((BE, D), lambda i: (i, 0)),
            pl.BlockSpec((BE, De), lambda i: (i, 0)),
            full((D, Dh)), full((De, Dh)), full((1, Dh)),
        ],
        out_specs=pl.BlockSpec((BE, P), lambda i: (i, 0)),
        out_shape=jax.ShapeDtypeStruct((E, P), jnp.float32),
    )(h1s, e2, We2aT, We2bT, be2)


def _tc_update(h0, mpos, mneg, WmT, bm):
    E, P = h0.shape
    Dh = WmT.shape[0]
    BE = 2048

    def body(h0_r, mp_r, mn_r, w_r, b_r, o_r):
        m = mp_r[...][:, :Dh] - mn_r[...][:, :Dh]
        v = _relu(h0_r[...][:, :Dh] + _dot(m, w_r[...]) + b_r[...])
        if P > Dh:
            v = jnp.concatenate([v, jnp.zeros((BE, P - Dh), jnp.float32)], 1)
        o_r[...] = v

    full = lambda shape: pl.BlockSpec(shape, lambda i: (0, 0))
    return pl.pallas_call(
        body,
        grid=(E // BE,),
        in_specs=[
            pl.BlockSpec((BE, P), lambda i: (i, 0)),
            pl.BlockSpec((BE, P), lambda i: (i, 0)),
            pl.BlockSpec((BE, P), lambda i: (i, 0)),
            full((Dh, Dh)), full((1, Dh)),
        ],
        out_specs=pl.BlockSpec((BE, P), lambda i: (i, 0)),
        out_shape=jax.ShapeDtypeStruct((E, P), jnp.float32),
    )(h0, mpos, mneg, WmT, bm)


def _tc_node_ln(xin, nm, WnaT, WnbT, bn, batchf, gamma, beta):
    N, D = xin.shape
    Do = WnaT.shape[1]
    Pm = nm.shape[1]
    G = _NUM_GRAPHS

    def body(x_r, nm_r, wa_r, wb_r, b_r, bat_r, g_r, be_r, o_r):
        n = _relu(_dot(x_r[...], wa_r[...]) + _dot(nm_r[...][:, :Do], wb_r[...])
                  + b_r[...])
        gids = lax.broadcasted_iota(jnp.int32, (N, G), 1).astype(jnp.float32)
        sel = (bat_r[...] == gids).astype(jnp.float32)
        cnt_nodes = jnp.sum(sel, axis=0, keepdims=True)
        cnt = jnp.maximum(cnt_nodes * float(Do), 1.0)
        rows = jnp.sum(n, axis=1, keepdims=True)
        gsum = lax.dot_general(sel, rows, (((0,), (0,)), ((), ())),
                               preferred_element_type=jnp.float32)
        mean_g = gsum / cnt.reshape(G, 1)
        mean_pn = _dot(sel, mean_g)
        xc = n - mean_pn
        rows2 = jnp.sum(xc * xc, axis=1, keepdims=True)
        vsum = lax.dot_general(sel, rows2, (((0,), (0,)), ((), ())),
                               preferred_element_type=jnp.float32)
        var_g = vsum / cnt.reshape(G, 1)
        var_pn = _dot(sel, var_g)
        ln = xc * lax.rsqrt(var_pn + 1e-5) * g_r[...] + be_r[...]
        o_r[...] = _elu(ln)

    full = lambda shape: pl.BlockSpec(shape, lambda i: (0, 0))
    return pl.pallas_call(
        body,
        grid=(1,),
        in_specs=[
            full((N, D)), full((N, Pm)),
            full((D, Do)), full((Do, Do)), full((1, Do)),
            full((N, 1)), full((1, Do)), full((1, Do)),
        ],
        out_specs=full((N, Do)),
        out_shape=jax.ShapeDtypeStruct((N, Do), jnp.float32),
    )(xin, nm, WnaT, WnbT, bn, batchf, gamma, beta)


def kernel(x, edge_index, edge_attr, batch, Wu1, bu1, Wu2, bu2, We1, be1,
           Wm1, bm1, Wn1, bn1, g1, b1, We2, be2, Wm2, bm2, Wn2, bn2, g2, b2):
    N, D = x.shape
    E = edge_index.shape[1]
    De = edge_attr.shape[1]
    Dh = Wm2.shape[0]

    src = edge_index[0].astype(jnp.int32)
    dst = edge_index[1].astype(jnp.int32)

    PADD = 256
    H = N + PADD + E
    key = src * N + dst
    krev = dst * N + src
    E2 = 2 * E
    comb = jnp.concatenate([key * 2, krev * 2 + 1])
    csort, cidx = lax.sort_key_val(comb, lax.iota(jnp.int32, E2))
    ckey = csort >> 1
    ctag = csort & 1
    is_new = jnp.concatenate([jnp.ones((1,), jnp.bool_), ckey[1:] != ckey[:-1]])
    nk0 = is_new & (ctag == 0)
    grun = jnp.cumsum(nk0.astype(jnp.int32)) - 1
    pos2 = lax.iota(jnp.int32, E2)
    zz = lax.cummax(jnp.where(is_new, pos2 * 2 + ctag, -1))
    found_run = (zz & 1) == 0
    orig = cidx - E
    eidx1 = jnp.where(ctag == 1, orig, E)
    val1 = jnp.where(found_run, (N + PADD) + grun, N + (orig & (_L - 1)))
    grev = jnp.zeros((E + 1,), jnp.int32).at[eidx1].set(val1)[:E]
    eidx0 = jnp.where(ctag == 0, cidx, E)
    inv = jnp.zeros((E + 1,), jnp.int32).at[eidx0].set(grun)[:E]

    tpos = dst
    tneg = (N + PADD) + inv
    gsrc = src

    as2d = lambda a: a.reshape(E // 128, 128)
    tpos2d, tneg2d = as2d(tpos), as2d(tneg)
    gsrc2d, grev2d = as2d(gsrc), as2d(grev)

    Wu1T, Wu2T = Wu1.T, Wu2.T
    We1aT, We1bT = We1[:, :D].T, We1[:, D:].T
    Wm1T = Wm1.T
    Wn1aT, Wn1bT = Wn1[:, :D].T, Wn1[:, D:].T
    We2aT, We2bT = We2[:, :D].T, We2[:, D:].T
    Wm2T = Wm2.T
    Wn2aT, Wn2bT = Wn2[:, :D].T, Wn2[:, D:].T
    row = lambda v: v.reshape(1, -1)
    batchf = batch.astype(jnp.float32).reshape(N, 1)

    xs = _sc_gather(x, gsrc2d)
    h0, e2 = _tc_prep1(xs, edge_attr, Wu1T, row(bu1), Wu2T, row(bu2),
                       We1aT, We1bT, row(be1))
    h = h0
    for _ in range(_T_STEPS):
        mpos, mneg = _sc_msg(h, tpos2d, tneg2d, gsrc2d, grev2d, H)
        h = _tc_update(h0, mpos, mneg, Wm1T, row(bm1))
    nm = _sc_nodemsg(h, tpos2d, N)
    h1 = _tc_node_ln(x, nm, Wn1aT, Wn1bT, row(bn1), batchf, row(g1), row(b1))

    h1s = _sc_gather(h1, gsrc2d)
    h02 = _tc_prep2(h1s, e2, We2aT, We2bT, row(be2), D)
    h = h02
    for _ in range(_T_STEPS):
        mpos, mneg = _sc_msg(h, tpos2d, tneg2d, gsrc2d, grev2d, H)
        h = _tc_update(h02, mpos, mneg, Wm2T, row(bm2))
    nm2 = _sc_nodemsg(h, tpos2d, N)
    h2 = _tc_node_ln(h1, nm2, Wn2aT, Wn2bT, row(bn2), batchf, row(g2), row(b2))
    return h2
